# baseline (speedup 1.0000x reference)
import functools

import jax
import jax.numpy as jnp
from jax.experimental import pallas as pl
from jax.experimental.pallas import tpu as pltpu

K = 8


def _router_kernel(x_ref, freq_ref, noise_ref, gw_ref, fgw_ref,
                   gates_ref, idx_ref, val_ref, *, spatial):
    pooled = jnp.sum(x_ref[...], axis=2) * (1.0 / spatial)
    logits = (
        jax.lax.dot(pooled, gw_ref[...], preferred_element_type=jnp.float32)
        + jax.lax.dot(freq_ref[...], fgw_ref[...],
                      preferred_element_type=jnp.float32)
        + noise_ref[...]
    )
    m = jnp.max(logits, axis=1, keepdims=True)
    e = jnp.exp(logits - m)
    probs = e / jnp.sum(e, axis=1, keepdims=True)

    bB, E = probs.shape
    lane = jax.lax.broadcasted_iota(jnp.int32, (bB, E), 1)
    work = probs
    keep = jnp.zeros((bB, E), dtype=jnp.bool_)
    vals = []
    idxs = []
    for _ in range(K):
        cur = jnp.max(work, axis=1, keepdims=True)
        cur_i = jnp.min(jnp.where(work == cur, lane, E), axis=1, keepdims=True)
        sel = lane == cur_i
        keep = jnp.logical_or(keep, sel)
        work = jnp.where(sel, -jnp.inf, work)
        vals.append(cur)
        idxs.append(cur_i)
    gates_ref[...] = jnp.where(keep, probs, 0.0)
    val_ref[...] = jnp.concatenate(vals, axis=1)
    idx_ref[...] = jnp.concatenate(idxs, axis=1)


def kernel(x, freq_emb, gate_w, freq_gate_w):
    B, DIM, H, W = x.shape
    FREQ = freq_emb.shape[1]
    E = gate_w.shape[0]
    S = H * W
    noise_std = 1.0 / E
    noise = jax.random.normal(jax.random.key(42), (B, E),
                              dtype=jnp.float32) * noise_std

    x3 = x.reshape(B, DIM, S)
    gw_t = gate_w.T
    fgw_t = freq_gate_w.T

    bB = 64
    grid = (B // bB,)

    gates, idxs, vals = pl.pallas_call(
        functools.partial(_router_kernel, spatial=float(S)),
        grid=grid,
        in_specs=[
            pl.BlockSpec((bB, DIM, S), lambda i: (i, 0, 0)),
            pl.BlockSpec((bB, FREQ), lambda i: (i, 0)),
            pl.BlockSpec((bB, E), lambda i: (i, 0)),
            pl.BlockSpec((DIM, E), lambda i: (0, 0)),
            pl.BlockSpec((FREQ, E), lambda i: (0, 0)),
        ],
        out_specs=[
            pl.BlockSpec((bB, E), lambda i: (i, 0)),
            pl.BlockSpec((bB, K), lambda i: (i, 0)),
            pl.BlockSpec((bB, K), lambda i: (i, 0)),
        ],
        out_shape=[
            jax.ShapeDtypeStruct((B, E), jnp.float32),
            jax.ShapeDtypeStruct((B, K), jnp.int32),
            jax.ShapeDtypeStruct((B, K), jnp.float32),
        ],
        compiler_params=pltpu.CompilerParams(
            dimension_semantics=("arbitrary",),
        ),
    )(x3, freq_emb, noise, gw_t, fgw_t)

    return (gates, idxs, vals, jnp.float32(0.0))


# speedup vs baseline: 4.2687x; 4.2687x over previous
import functools

import jax
import jax.numpy as jnp
from jax.experimental import pallas as pl
from jax.experimental.pallas import tpu as pltpu

K = 8


def _router_kernel(x_ref, freq_ref, noise_ref, gw_ref, fgw_ref,
                   gates_ref, idx_ref, val_ref, *, spatial):
    pooled = jnp.sum(x_ref[...], axis=0) * (1.0 / spatial)
    logits = (
        jax.lax.dot(pooled, gw_ref[...], preferred_element_type=jnp.float32)
        + jax.lax.dot(freq_ref[...], fgw_ref[...],
                      preferred_element_type=jnp.float32)
        + noise_ref[...]
    )
    m = jnp.max(logits, axis=1, keepdims=True)
    e = jnp.exp(logits - m)
    probs = e / jnp.sum(e, axis=1, keepdims=True)

    bB, E = probs.shape
    lane = jax.lax.broadcasted_iota(jnp.int32, (bB, E), 1)
    work = probs
    keep = jnp.zeros((bB, E), dtype=jnp.bool_)
    vals = []
    idxs = []
    for _ in range(K):
        cur = jnp.max(work, axis=1, keepdims=True)
        cur_i = jnp.min(jnp.where(work == cur, lane, E), axis=1, keepdims=True)
        sel = lane == cur_i
        keep = jnp.logical_or(keep, sel)
        work = jnp.where(sel, -jnp.inf, work)
        vals.append(cur)
        idxs.append(cur_i)
    gates_ref[...] = jnp.where(keep, probs, 0.0)
    val_ref[...] = jnp.concatenate(vals, axis=1)
    idx_ref[...] = jnp.concatenate(idxs, axis=1)


def kernel(x, freq_emb, gate_w, freq_gate_w):
    B, DIM, H, W = x.shape
    FREQ = freq_emb.shape[1]
    E = gate_w.shape[0]
    S = H * W
    noise_std = 1.0 / E
    noise = jax.random.normal(jax.random.key(42), (B, E),
                              dtype=jnp.float32) * noise_std

    x_t = x.transpose(2, 3, 0, 1).reshape(S, B, DIM)
    gw_t = gate_w.T
    fgw_t = freq_gate_w.T

    bB = 64
    grid = (B // bB,)

    gates, idxs, vals = pl.pallas_call(
        functools.partial(_router_kernel, spatial=float(S)),
        grid=grid,
        in_specs=[
            pl.BlockSpec((S, bB, DIM), lambda i: (0, i, 0)),
            pl.BlockSpec((bB, FREQ), lambda i: (i, 0)),
            pl.BlockSpec((bB, E), lambda i: (i, 0)),
            pl.BlockSpec((DIM, E), lambda i: (0, 0)),
            pl.BlockSpec((FREQ, E), lambda i: (0, 0)),
        ],
        out_specs=[
            pl.BlockSpec((bB, E), lambda i: (i, 0)),
            pl.BlockSpec((bB, K), lambda i: (i, 0)),
            pl.BlockSpec((bB, K), lambda i: (i, 0)),
        ],
        out_shape=[
            jax.ShapeDtypeStruct((B, E), jnp.float32),
            jax.ShapeDtypeStruct((B, K), jnp.int32),
            jax.ShapeDtypeStruct((B, K), jnp.float32),
        ],
        compiler_params=pltpu.CompilerParams(
            dimension_semantics=("arbitrary",),
        ),
    )(x_t, freq_emb, noise, gw_t, fgw_t)

    return (gates, idxs, vals, jnp.float32(0.0))


# speedup vs baseline: 4.3062x; 1.0088x over previous
import functools

import jax
import jax.numpy as jnp
from jax.experimental import pallas as pl
from jax.experimental.pallas import tpu as pltpu

K = 8


def _router_kernel(x_ref, freq_ref, noise_ref, gw_ref, fgw_ref,
                   gates_ref, idx_ref, val_ref, *, spatial):
    pooled = jnp.sum(x_ref[...], axis=0) * (1.0 / spatial)
    logits = (
        jax.lax.dot(pooled, gw_ref[...], preferred_element_type=jnp.float32)
        + jax.lax.dot(freq_ref[...], fgw_ref[...],
                      preferred_element_type=jnp.float32)
        + noise_ref[...]
    )
    m = jnp.max(logits, axis=1, keepdims=True)
    e = jnp.exp(logits - m)
    probs = e / jnp.sum(e, axis=1, keepdims=True)

    bB, E = probs.shape
    lane = jax.lax.broadcasted_iota(jnp.int32, (bB, E), 1)
    work = probs
    keep = jnp.zeros((bB, E), dtype=jnp.bool_)
    vals = []
    idxs = []
    for _ in range(K):
        cur = jnp.max(work, axis=1, keepdims=True)
        cur_i = jnp.min(jnp.where(work == cur, lane, E), axis=1, keepdims=True)
        sel = lane == cur_i
        keep = jnp.logical_or(keep, sel)
        work = jnp.where(sel, -jnp.inf, work)
        vals.append(cur)
        idxs.append(cur_i)
    gates_ref[...] = jnp.where(keep, probs, 0.0)
    val_ref[...] = jnp.concatenate(vals, axis=1)
    idx_ref[...] = jnp.concatenate(idxs, axis=1)


def kernel(x, freq_emb, gate_w, freq_gate_w):
    B, DIM, H, W = x.shape
    FREQ = freq_emb.shape[1]
    E = gate_w.shape[0]
    S = H * W
    noise_std = 1.0 / E
    noise = jax.random.normal(jax.random.key(42), (B, E),
                              dtype=jnp.float32) * noise_std

    x_t = x.transpose(2, 3, 0, 1).reshape(S, B, DIM)
    gw_t = gate_w.T
    fgw_t = freq_gate_w.T

    bB = 128
    grid = (B // bB,)

    gates, idxs, vals = pl.pallas_call(
        functools.partial(_router_kernel, spatial=float(S)),
        grid=grid,
        in_specs=[
            pl.BlockSpec((S, bB, DIM), lambda i: (0, i, 0)),
            pl.BlockSpec((bB, FREQ), lambda i: (i, 0)),
            pl.BlockSpec((bB, E), lambda i: (i, 0)),
            pl.BlockSpec((DIM, E), lambda i: (0, 0)),
            pl.BlockSpec((FREQ, E), lambda i: (0, 0)),
        ],
        out_specs=[
            pl.BlockSpec((bB, E), lambda i: (i, 0)),
            pl.BlockSpec((bB, K), lambda i: (i, 0)),
            pl.BlockSpec((bB, K), lambda i: (i, 0)),
        ],
        out_shape=[
            jax.ShapeDtypeStruct((B, E), jnp.float32),
            jax.ShapeDtypeStruct((B, K), jnp.int32),
            jax.ShapeDtypeStruct((B, K), jnp.float32),
        ],
        compiler_params=pltpu.CompilerParams(
            dimension_semantics=("arbitrary",),
        ),
    )(x_t, freq_emb, noise, gw_t, fgw_t)

    return (gates, idxs, vals, jnp.float32(0.0))
